# baseline (speedup 1.0000x reference)
import jax
import jax.numpy as jnp
from jax import lax
from jax.experimental import pallas as pl
from jax.experimental.pallas import tpu as pltpu
from jax.experimental.pallas import tpu_sc as plsc

B = 16384
C = 6
D = 64
NW = 32
B_W = B // NW
CHUNK_B = 64
N_CHUNKS = B_W // CHUNK_B
CHUNK_ROWS = CHUNK_B * C
GATHER_N = 128
G_PER_CHUNK = CHUNK_ROWS // GATHER_N


def _sc_body(ctr_ids, ctx_ids, ctr_table, ctx_table, out,
             ctr_idx_v, ctx_idx_v, ctr_buf, ctx_bufs, scores_v,
             sem_ctr, sem_a, sem_b):
    nc = 2
    wid = lax.axis_index("s") * nc + lax.axis_index("c")

    pltpu.sync_copy(ctr_ids.at[pl.ds(wid * (B_W // 128), B_W // 128)],
                    ctr_idx_v)
    pltpu.sync_copy(ctx_ids.at[pl.ds(wid * (B_W * C // 128), B_W * C // 128)],
                    ctx_idx_v)

    for j in range(B_W // GATHER_N):
        pltpu.make_async_copy(
            ctr_table.at[ctr_idx_v.at[j]],
            ctr_buf.at[pl.ds(j * GATHER_N, GATHER_N)], sem_ctr).start()
    for j in range(B_W // GATHER_N):
        pltpu.make_async_copy(
            ctr_table.at[pl.ds(0, GATHER_N)],
            ctr_buf.at[pl.ds(j * GATHER_N, GATHER_N)], sem_ctr).wait()

    def issue_ctx(chunk, buf, sem):
        for j in range(G_PER_CHUNK):
            pltpu.make_async_copy(
                ctx_table.at[ctx_idx_v.at[chunk * G_PER_CHUNK + j]],
                buf.at[pl.ds(j * GATHER_N, GATHER_N)], sem).start()

    def drain_ctx(buf, sem):
        for j in range(G_PER_CHUNK):
            pltpu.make_async_copy(
                ctx_table.at[pl.ds(0, GATHER_N)],
                buf.at[pl.ds(j * GATHER_N, GATHER_N)], sem).wait()

    lane = lax.iota(jnp.int32, 16)

    def compute_chunk(chunk, buf):
        def group(g, _):
            b_in_chunk = g * 16 + lane
            rows_ctr = chunk * CHUNK_B + b_in_chunk
            rows_ctx = [b_in_chunk * C + c for c in range(C)]
            accs = [jnp.zeros((16,), jnp.float32) for _ in range(C)]
            for d in range(D):
                cold = jnp.full((16,), d, jnp.int32)
                ctr_v = plsc.load_gather(ctr_buf, [rows_ctr, cold])
                for c in range(C):
                    ctx_v = plsc.load_gather(buf, [rows_ctx[c], cold])
                    accs[c] = accs[c] + ctx_v * ctr_v
            b_w = chunk * CHUNK_B + g * 16 + lane
            for c in range(C):
                s = jnp.minimum(jnp.maximum(accs[c], -10.0), 10.0)
                plsc.store_scatter(scores_v, [b_w * C + c], s)
            return ()

        lax.fori_loop(0, CHUNK_B // 16, group, (), unroll=False)

    issue_ctx(0, ctx_bufs[0], sem_a)

    def chunk_pair(k2, _):
        issue_ctx(k2 + 1, ctx_bufs[1], sem_b)
        drain_ctx(ctx_bufs[0], sem_a)
        compute_chunk(k2, ctx_bufs[0])

        @pl.when(k2 + 2 < N_CHUNKS)
        def _():
            issue_ctx(k2 + 2, ctx_bufs[0], sem_a)

        drain_ctx(ctx_bufs[1], sem_b)
        compute_chunk(k2 + 1, ctx_bufs[1])
        return ()

    lax.fori_loop(0, N_CHUNKS // 2, lambda i, c: chunk_pair(i * 2, c), (),
                  unroll=False)

    pltpu.sync_copy(scores_v, out.at[pl.ds(wid * B_W * C, B_W * C)])


@jax.jit
def _scores(center_ids2d, context_ids2d, center_table, context_table):
    mesh = plsc.VectorSubcoreMesh(core_axis_name="c", subcore_axis_name="s")
    flat = pl.kernel(
        _sc_body,
        out_type=jax.ShapeDtypeStruct((B * C,), jnp.float32),
        mesh=mesh,
        compiler_params=pltpu.CompilerParams(needs_layout_passes=False,
                                             use_tc_tiling_on_sc=False),
        scratch_types=[
            pltpu.VMEM((B_W // 128, 128), jnp.int32),
            pltpu.VMEM((B_W * C // 128, 128), jnp.int32),
            pltpu.VMEM((B_W, D), jnp.float32),
            [pltpu.VMEM((CHUNK_ROWS, D), jnp.float32),
             pltpu.VMEM((CHUNK_ROWS, D), jnp.float32)],
            pltpu.VMEM((B_W * C,), jnp.float32),
            pltpu.SemaphoreType.DMA,
            pltpu.SemaphoreType.DMA,
            pltpu.SemaphoreType.DMA,
        ],
    )(center_ids2d, context_ids2d, center_table, context_table)
    return flat.reshape(B, C)


def kernel(center_ids, context_ids, center_table, context_table):
    ctr2d = center_ids.reshape(B // 128, 128)
    ctx2d = context_ids.reshape(B * C // 128, 128)
    return _scores(ctr2d, ctx2d, center_table, context_table)


# speedup vs baseline: 1.0136x; 1.0136x over previous
import jax
import jax.numpy as jnp
from jax import lax
from jax.experimental import pallas as pl
from jax.experimental.pallas import tpu as pltpu
from jax.experimental.pallas import tpu_sc as plsc

B = 16384
C = 6
D = 64
NW = 32
B_W = B // NW
CHUNK_B = 32
N_CHUNKS = B_W // CHUNK_B
CHUNK_ROWS = CHUNK_B * C
CTX_GN = 96
D_UNROLL = 16


def _sc_body(ctr_ids, ctx_ids, ctr_tab2, ctx_tab2, out,
             ctr_idx_v, ctx_idx_v, ctr_pair_v, ctx_pair_v,
             ctr_buf, ctx_bufs, scores_v, sem_ctr, sem_a, sem_b):
    nc = 2
    wid = lax.axis_index("s") * nc + lax.axis_index("c")

    pltpu.sync_copy(ctr_ids.at[pl.ds(wid * (B_W // 128), B_W // 128)],
                    ctr_idx_v)
    pltpu.sync_copy(ctx_ids.at[pl.ds(wid * (B_W * C // 128), B_W * C // 128)],
                    ctx_idx_v)

    for r in range(B_W // 128):
        for k in range(128 // 16):
            ctr_pair_v[r, pl.ds(k * 16, 16)] = (
                ctr_idx_v[r, pl.ds(k * 16, 16)] >> 1)

    def repack(r, _):
        for j in range(6):
            i = r * 6 + j
            rr = i >> 3
            cc = (i & 7) * 16
            ctx_pair_v[r, pl.ds(j * 16, 16)] = (
                ctx_idx_v[rr, pl.ds(cc, 16)] >> 1)
        return ()

    lax.fori_loop(0, B_W * C // 96, repack, (), unroll=False)

    for j in range(B_W // 128):
        pltpu.make_async_copy(
            ctr_tab2.at[ctr_pair_v.at[j]],
            ctr_buf.at[pl.ds(j * 128, 128)], sem_ctr).start()
    for j in range(B_W // 128):
        pltpu.make_async_copy(
            ctr_tab2.at[pl.ds(0, 128)],
            ctr_buf.at[pl.ds(j * 128, 128)], sem_ctr).wait()

    def issue_ctx(chunk, buf, sem):
        for j in range(2):
            pltpu.make_async_copy(
                ctx_tab2.at[ctx_pair_v.at[chunk * 2 + j]],
                buf.at[pl.ds(j * CTX_GN, CTX_GN)], sem).start()

    def drain_ctx(buf, sem):
        for j in range(2):
            pltpu.make_async_copy(
                ctx_tab2.at[pl.ds(0, CTX_GN)],
                buf.at[pl.ds(j * CTX_GN, CTX_GN)], sem).wait()

    lane = lax.iota(jnp.int32, 16)

    def compute_chunk(chunk, buf):
        def group(g, _):
            b_in_chunk = g * 16 + lane
            b_w = chunk * CHUNK_B + b_in_chunk
            raw_ctr = plsc.load_gather(ctr_idx_v, [b_w >> 7, b_w & 127])
            col0_ctr = (raw_ctr & 1) * 64
            rows_ctx = []
            col0_ctx = []
            for c in range(C):
                q = b_w * C + c
                raw = plsc.load_gather(ctx_idx_v, [q >> 7, q & 127])
                rows_ctx.append(b_in_chunk * C + c)
                col0_ctx.append((raw & 1) * 64)

            def dstep(t, carry):
                accs = list(carry)
                for dd in range(D_UNROLL):
                    d = t * D_UNROLL + dd
                    ctr_v = plsc.load_gather(ctr_buf, [b_w, col0_ctr + d])
                    for c in range(C):
                        ctx_v = plsc.load_gather(
                            buf, [rows_ctx[c], col0_ctx[c] + d])
                        accs[c] = accs[c] + ctx_v * ctr_v
                return tuple(accs)

            zeros = tuple(jnp.zeros((16,), jnp.float32) for _ in range(C))
            accs = lax.fori_loop(0, D // D_UNROLL, dstep, zeros, unroll=False)
            for c in range(C):
                s = jnp.minimum(jnp.maximum(accs[c], -10.0), 10.0)
                plsc.store_scatter(scores_v, [b_w * C + c], s)
            return ()

        lax.fori_loop(0, CHUNK_B // 16, group, (), unroll=False)

    issue_ctx(0, ctx_bufs[0], sem_a)

    def chunk_pair(k2, _):
        issue_ctx(k2 + 1, ctx_bufs[1], sem_b)
        drain_ctx(ctx_bufs[0], sem_a)
        compute_chunk(k2, ctx_bufs[0])

        @pl.when(k2 + 2 < N_CHUNKS)
        def _():
            issue_ctx(k2 + 2, ctx_bufs[0], sem_a)

        drain_ctx(ctx_bufs[1], sem_b)
        compute_chunk(k2 + 1, ctx_bufs[1])
        return ()

    lax.fori_loop(0, N_CHUNKS // 2, lambda i, c: chunk_pair(i * 2, c), (),
                  unroll=False)

    pltpu.sync_copy(scores_v, out.at[pl.ds(wid * B_W * C, B_W * C)])


@jax.jit
def _scores(center_ids2d, context_ids2d, ctr_tab2, ctx_tab2):
    mesh = plsc.VectorSubcoreMesh(core_axis_name="c", subcore_axis_name="s")
    flat = pl.kernel(
        _sc_body,
        out_type=jax.ShapeDtypeStruct((B * C,), jnp.float32),
        mesh=mesh,
        compiler_params=pltpu.CompilerParams(needs_layout_passes=False),
        scratch_types=[
            pltpu.VMEM((B_W // 128, 128), jnp.int32),
            pltpu.VMEM((B_W * C // 128, 128), jnp.int32),
            pltpu.VMEM((B_W // 128, 128), jnp.int32),
            pltpu.VMEM((B_W * C // 96, 96), jnp.int32),
            pltpu.VMEM((B_W, 128), jnp.float32),
            [pltpu.VMEM((CHUNK_ROWS, 128), jnp.float32),
             pltpu.VMEM((CHUNK_ROWS, 128), jnp.float32)],
            pltpu.VMEM((B_W * C,), jnp.float32),
            pltpu.SemaphoreType.DMA,
            pltpu.SemaphoreType.DMA,
            pltpu.SemaphoreType.DMA,
        ],
    )(center_ids2d, context_ids2d, ctr_tab2, ctx_tab2)
    return flat.reshape(B, C)


def kernel(center_ids, context_ids, center_table, context_table):
    ctr2d = center_ids.reshape(B // 128, 128)
    ctx2d = context_ids.reshape(B * C // 128, 128)
    ctr_tab2 = center_table.reshape(-1, 128)
    ctx_tab2 = context_table.reshape(-1, 128)
    return _scores(ctr2d, ctx2d, ctr_tab2, ctx_tab2)
